# baseline (speedup 1.0000x reference)
import functools

import jax
import jax.numpy as jnp
from jax import lax
from jax.experimental import pallas as pl
from jax.experimental.pallas import tpu as pltpu
from jax.experimental.pallas import tpu_sc as plsc

NUM_CLASSES = 100000
FEAT_DIM = 64
BATCH = 16384

_INFO = plsc.get_sparse_core_info()
_NC = _INFO.num_cores
_NS = _INFO.num_subcores
_NW = _NC * _NS
_L = _INFO.num_lanes

_B_PER_W = BATCH // _NW
_CHUNK = 128
_NCHUNK = _B_PER_W // _CHUNK


def _body(xt_hbm, lab_hbm, cen_hbm, out_hbm, idx_v, x_v, c_v, acc_v,
          s0, s1, s2, s3, xsem):
    wid = lax.axis_index("s") * _NC + lax.axis_index("c")
    base = wid * _NCHUNK

    pltpu.sync_copy(lab_hbm.at[pl.ds(base, _NCHUNK)], idx_v)
    xcopy = pltpu.async_copy(
        xt_hbm.at[:, pl.ds(base * _CHUNK, _B_PER_W)], x_v, xsem)

    sems = (s0, s1, s2, s3)
    gathers = [
        pltpu.async_copy(cen_hbm.at[idx_v.at[j]], c_v.at[j], sems[j])
        for j in range(_NCHUNK)
    ]
    xcopy.wait()

    lane_iota = jax.lax.iota(jnp.int32, _L)
    zero = jnp.zeros((_L,), jnp.float32)

    def chunk_loop(j):
        def k_body(k, accs):
            a0, a1, a2, a3 = accs
            s_off = pl.multiple_of(k * _L, _L)
            row_idx = lane_iota + s_off
            j_idx = jnp.full((_L,), j, jnp.int32)
            for f in range(FEAT_DIM):
                xv = x_v[f, pl.ds(base_s + s_off, _L)]
                cv = plsc.load_gather(
                    c_v, [j_idx, row_idx, jnp.full((_L,), f, jnp.int32)])
                d = xv - cv
                if f % 4 == 0:
                    a0 = a0 + d * d
                elif f % 4 == 1:
                    a1 = a1 + d * d
                elif f % 4 == 2:
                    a2 = a2 + d * d
                else:
                    a3 = a3 + d * d
            return (a0, a1, a2, a3)

        base_s = j * _CHUNK
        return k_body

    accs = (zero, zero, zero, zero)
    for j in range(_NCHUNK):
        gathers[j].wait()
        accs = lax.fori_loop(0, _CHUNK // _L, chunk_loop(j), accs)

    acc_v[...] = accs[0] + accs[1] + accs[2] + accs[3]
    pltpu.sync_copy(acc_v, out_hbm.at[wid])


@jax.jit
def _center_loss(x, labels, centers):
    xt = x.T
    lab = labels.astype(jnp.int32).reshape(_NW * _NCHUNK, _CHUNK)
    run = functools.partial(
        pl.kernel,
        out_type=jax.ShapeDtypeStruct((_NW, _L), jnp.float32),
        mesh=plsc.VectorSubcoreMesh(core_axis_name="c", subcore_axis_name="s"),
        compiler_params=pltpu.CompilerParams(
            use_tc_tiling_on_sc=False, needs_layout_passes=False),
        scratch_types=[
            pltpu.VMEM((_NCHUNK, _CHUNK), jnp.int32),
            pltpu.VMEM((FEAT_DIM, _B_PER_W), jnp.float32),
            pltpu.VMEM((_NCHUNK, _CHUNK, FEAT_DIM), jnp.float32),
            pltpu.VMEM((_L,), jnp.float32),
            pltpu.SemaphoreType.DMA,
            pltpu.SemaphoreType.DMA,
            pltpu.SemaphoreType.DMA,
            pltpu.SemaphoreType.DMA,
            pltpu.SemaphoreType.DMA,
        ],
    )(_body)
    partials = run(xt, lab, centers)
    return jnp.sum(partials) / 2.0 / BATCH


def kernel(x, labels, centers):
    return _center_loss(x, labels, centers)


# speedup vs baseline: 1.0935x; 1.0935x over previous
import functools

import jax
import jax.numpy as jnp
from jax import lax
from jax.experimental import pallas as pl
from jax.experimental.pallas import tpu as pltpu
from jax.experimental.pallas import tpu_sc as plsc

NUM_CLASSES = 100000
FEAT_DIM = 64
BATCH = 16384

_INFO = plsc.get_sparse_core_info()
_NC = _INFO.num_cores
_NS = _INFO.num_subcores
_NW = _NC * _NS
_L = _INFO.num_lanes

_B_PER_W = BATCH // _NW
_CHUNK = 128
_NCHUNK = _B_PER_W // _CHUNK
_PAIR = 2 * FEAT_DIM


def _sc_body(x_hbm, lab_hbm, cen_hbm, out_hbm, idx_v, lab_s, x_v, c_v, acc_v,
             s0, s1, s2, s3, xsem):
    wid = lax.axis_index("s") * _NC + lax.axis_index("c")
    base = wid * _NCHUNK

    pltpu.sync_copy(lab_hbm.at[pl.ds(base, _NCHUNK)], idx_v)
    xcopy = pltpu.async_copy(
        x_hbm.at[pl.ds(base * (_CHUNK // 2), _B_PER_W // 2)], x_v, xsem)

    for j in range(_NCHUNK):
        for v in range(_CHUNK // _L):
            sl = pl.ds(v * _L, _L)
            lab = idx_v[j, sl]
            lab_s[j, sl] = (lab & 1) * FEAT_DIM
            idx_v[j, sl] = lax.shift_right_logical(lab, 1)

    sems = (s0, s1, s2, s3)
    gathers = [
        pltpu.async_copy(cen_hbm.at[idx_v.at[j]], c_v.at[j], sems[j])
        for j in range(_NCHUNK)
    ]
    xcopy.wait()

    def make_group_body(j):
        def group_body(k, accs):
            a0, a1, a2, a3 = accs
            offs = lab_s[j, pl.ds(k * _L, _L)]
            for t in range(_L):
                s = k * _L + t
                xr = j * (_CHUNK // 2) + lax.div(s, 2)
                xoff = FEAT_DIM * (t % 2)
                coff = pl.multiple_of(offs[t], FEAT_DIM)
                d0 = (x_v[xr, pl.ds(xoff, _L)]
                      - c_v[j, s, pl.ds(coff, _L)])
                d1 = (x_v[xr, pl.ds(xoff + _L, _L)]
                      - c_v[j, s, pl.ds(coff + _L, _L)])
                d2 = (x_v[xr, pl.ds(xoff + 2 * _L, _L)]
                      - c_v[j, s, pl.ds(coff + 2 * _L, _L)])
                d3 = (x_v[xr, pl.ds(xoff + 3 * _L, _L)]
                      - c_v[j, s, pl.ds(coff + 3 * _L, _L)])
                a0 = a0 + d0 * d0
                a1 = a1 + d1 * d1
                a2 = a2 + d2 * d2
                a3 = a3 + d3 * d3
            return (a0, a1, a2, a3)
        return group_body

    zero = jnp.zeros((_L,), jnp.float32)
    accs = (zero, zero, zero, zero)
    for j in range(_NCHUNK):
        gathers[j].wait()
        accs = lax.fori_loop(0, _CHUNK // _L, make_group_body(j), accs)

    acc_v[...] = accs[0] + accs[1] + accs[2] + accs[3]
    pltpu.sync_copy(acc_v, out_hbm.at[wid])


@jax.jit
def _center_loss(x, labels, centers):
    x2 = x.reshape(BATCH // 2, _PAIR)
    cen2 = centers.reshape(NUM_CLASSES // 2, _PAIR)
    lab = labels.astype(jnp.int32).reshape(_NW * _NCHUNK, _CHUNK)
    run = functools.partial(
        pl.kernel,
        out_type=jax.ShapeDtypeStruct((_NW, _L), jnp.float32),
        mesh=plsc.VectorSubcoreMesh(core_axis_name="c", subcore_axis_name="s"),
        compiler_params=pltpu.CompilerParams(use_tc_tiling_on_sc=True),
        scratch_types=[
            pltpu.VMEM((_NCHUNK, _CHUNK), jnp.int32),
            pltpu.VMEM((_NCHUNK, _CHUNK), jnp.int32),
            pltpu.VMEM((_B_PER_W // 2, _PAIR), jnp.float32),
            pltpu.VMEM((_NCHUNK, _CHUNK, _PAIR), jnp.float32),
            pltpu.VMEM((_L,), jnp.float32),
            pltpu.SemaphoreType.DMA,
            pltpu.SemaphoreType.DMA,
            pltpu.SemaphoreType.DMA,
            pltpu.SemaphoreType.DMA,
            pltpu.SemaphoreType.DMA,
        ],
    )(_sc_body)
    partials = run(x2, lab, cen2)
    return jnp.sum(partials) / 2.0 / BATCH


def kernel(x, labels, centers):
    return _center_loss(x, labels, centers)
